# Initial kernel scaffold; baseline (speedup 1.0000x reference)
import functools

import jax
import jax.numpy as jnp
from jax import lax
from jax.experimental import pallas as pl
from jax.experimental.pallas import tpu as pltpu
from jax.experimental.pallas import tpu_sc as plsc

N = 10000
E = 320000
D_IN = 128
D_H = 128
D_H2 = 64

NC = 2
NS = 16
NW = NC * NS
EPW = E // NW
CB = 80
CHUNKS = EPW // CB
NPAD = 10240
RPT = NPAD // NS


def _make_agg(D):
    mesh = plsc.VectorSubcoreMesh(
        core_axis_name="c", subcore_axis_name="s", num_cores=NC, num_subcores=NS
    )

    @functools.partial(
        pl.kernel,
        out_type=jax.ShapeDtypeStruct((NC, NPAD, D), jnp.float32),
        mesh=mesh,
        scratch_types=[
            pltpu.VMEM((CHUNKS, CB), jnp.int32),
            pltpu.VMEM((CHUNKS, CB), jnp.int32),
            pltpu.VMEM((CB, D), jnp.float32),
            pltpu.VMEM_SHARED((NPAD, D), jnp.float32),
            pltpu.SemaphoreType.DMA,
        ],
    )
    def agg(y_hbm, src_hbm, dst_hbm, zero_hbm, out_hbm, src_v, dst_v, rows_v,
            acc_sh, sem):
        c = lax.axis_index("c")
        s = lax.axis_index("s")
        w = c * NS + s
        pltpu.sync_copy(src_hbm.at[w], src_v)
        pltpu.sync_copy(dst_hbm.at[w], dst_v)
        r0 = s * RPT
        pltpu.sync_copy(zero_hbm, acc_sh.at[pl.ds(r0, RPT)])
        plsc.subcore_barrier()

        def body(j, carry):
            pltpu.async_copy(y_hbm.at[src_v.at[j]], rows_v, sem).wait()
            pltpu.sync_copy(rows_v, acc_sh.at[dst_v.at[j]], add=True)
            return carry

        lax.fori_loop(0, CHUNKS, body, 0)
        plsc.subcore_barrier()
        pltpu.sync_copy(acc_sh.at[pl.ds(r0, RPT)],
                        out_hbm.at[c].at[pl.ds(r0, RPT)])

    return agg


_agg1 = _make_agg(1)
_agg128 = _make_agg(128)
_agg64 = _make_agg(64)


_RB = 1000
_GRID = N // _RB


def _part_spec(core, d):
    return pl.BlockSpec((1, _RB, d), lambda i, c=core: (c, i, 0))


def _row_spec(d):
    return pl.BlockSpec((_RB, d), lambda i: (i, 0))


def _full_spec(shape):
    return pl.BlockSpec(shape, lambda i: tuple(0 for _ in shape))


def _k1_body(d0, d1, x, w1, y_ref, dis_ref):
    deg = d0[0] + d1[0] + 1.0
    dis = lax.rsqrt(deg)
    dis_ref[...] = dis
    y_ref[...] = dis * jnp.dot(x[...], w1[...],
                               preferred_element_type=jnp.float32)


def _k2_body(p0, p1, y1, dis, b1, w2, y2_ref):
    h = jnp.maximum(dis[...] * (p0[0] + p1[0] + y1[...]) + b1[...], 0.0)
    y2_ref[...] = dis[...] * jnp.dot(h, w2[...],
                                     preferred_element_type=jnp.float32)


def _k3_body(q0, q1, y2, dis, b2, z_ref):
    h = jnp.maximum(dis[...] * (q0[0] + q1[0] + y2[...]) + b2[...], 0.0)
    z_ref[...] = dis[...] * h


def _k4_body(r0, r1, z, dis, w3, b3, out_ref):
    t = r0[0] + r1[0] + z[...]
    out_ref[...] = dis[...] * jnp.dot(t, w3[...],
                                      preferred_element_type=jnp.float32) + b3[...]


def kernel(x, edge_index, W1, b1, W2, b2, W3, b3):
    f32 = jnp.float32
    src = edge_index[0].reshape(NW, CHUNKS, CB)
    dst = edge_index[1].reshape(NW, CHUNKS, CB)
    ones = jnp.ones((N, 1), f32)
    z1 = jnp.zeros((RPT, 1), f32)
    z64 = jnp.zeros((RPT, 64), f32)
    z128 = jnp.zeros((RPT, 128), f32)
    b1r = b1.reshape(1, D_H)
    b2r = b2.reshape(1, D_H2)
    b3r = b3.reshape(1, 1)

    degp = _agg1(ones, src, dst, z1)
    y1, dis = pl.pallas_call(
        _k1_body,
        grid=(_GRID,),
        in_specs=[_part_spec(0, 1), _part_spec(1, 1), _row_spec(D_IN),
                  _full_spec((D_IN, D_H))],
        out_specs=[_row_spec(D_H), _row_spec(1)],
        out_shape=[jax.ShapeDtypeStruct((N, D_H), f32),
                   jax.ShapeDtypeStruct((N, 1), f32)],
    )(degp, degp, x, W1)

    p = _agg128(y1, src, dst, z128)
    y2 = pl.pallas_call(
        _k2_body,
        grid=(_GRID,),
        in_specs=[_part_spec(0, D_H), _part_spec(1, D_H), _row_spec(D_H),
                  _row_spec(1), _full_spec((1, D_H)), _full_spec((D_H, D_H2))],
        out_specs=_row_spec(D_H2),
        out_shape=jax.ShapeDtypeStruct((N, D_H2), f32),
    )(p, p, y1, dis, b1r, W2)

    q = _agg64(y2, src, dst, z64)
    z = pl.pallas_call(
        _k3_body,
        grid=(_GRID,),
        in_specs=[_part_spec(0, D_H2), _part_spec(1, D_H2), _row_spec(D_H2),
                  _row_spec(1), _full_spec((1, D_H2))],
        out_specs=_row_spec(D_H2),
        out_shape=jax.ShapeDtypeStruct((N, D_H2), f32),
    )(q, q, y2, dis, b2r)

    r = _agg64(z, src, dst, z64)
    out = pl.pallas_call(
        _k4_body,
        grid=(_GRID,),
        in_specs=[_part_spec(0, D_H2), _part_spec(1, D_H2), _row_spec(D_H2),
                  _row_spec(1), _full_spec((D_H2, 1)), _full_spec((1, 1))],
        out_specs=_row_spec(1),
        out_shape=jax.ShapeDtypeStruct((N, 1), f32),
    )(r, r, z, dis, W3, b3r)
    return out


# speedup vs baseline: 12.7114x; 12.7114x over previous
import functools

import jax
import jax.numpy as jnp
from jax import lax
from jax.experimental import pallas as pl
from jax.experimental.pallas import tpu as pltpu
from jax.experimental.pallas import tpu_sc as plsc

N = 10000
E = 320000
D_IN = 128
D_H = 128
D_H2 = 64
DDEG = 8

NC = 2
NS = 16
NW = NC * NS
EPW = E // NW
CB = 80
CHUNKS = EPW // CB
NPAD = 10240
RPT = NPAD // NS

_SC_MESH = dict(core_axis_name="c", subcore_axis_name="s",
                num_cores=NC, num_subcores=NS)
_SC_PARAMS = pltpu.CompilerParams(use_tc_tiling_on_sc=False)


def _make_agg(D):
    @functools.partial(
        pl.kernel,
        out_type=jax.ShapeDtypeStruct((NC, NPAD, D), jnp.float32),
        mesh=plsc.VectorSubcoreMesh(**_SC_MESH),
        scratch_types=[
            pltpu.VMEM((CB,), jnp.int32),
            pltpu.VMEM((CB,), jnp.int32),
            pltpu.VMEM((CB, D), jnp.float32),
            pltpu.VMEM_SHARED((NPAD, D), jnp.float32),
            pltpu.SemaphoreType.DMA,
        ],
        compiler_params=_SC_PARAMS,
    )
    def agg(y_hbm, src_hbm, dst_hbm, zero_hbm, out_hbm, src_v, dst_v, rows_v,
            acc_sh, sem):
        c = lax.axis_index("c")
        s = lax.axis_index("s")
        w = c * NS + s
        r0 = s * RPT
        pltpu.sync_copy(zero_hbm, acc_sh.at[pl.ds(r0, RPT)])
        plsc.subcore_barrier()

        def body(j, carry):
            pltpu.sync_copy(src_hbm.at[w, j], src_v)
            pltpu.sync_copy(dst_hbm.at[w, j], dst_v)
            pltpu.async_copy(y_hbm.at[src_v], rows_v, sem).wait()
            pltpu.sync_copy(rows_v, acc_sh.at[dst_v], add=True)
            return carry

        lax.fori_loop(0, CHUNKS, body, 0)
        plsc.subcore_barrier()
        pltpu.sync_copy(acc_sh.at[pl.ds(r0, RPT)],
                        out_hbm.at[c].at[pl.ds(r0, RPT)])

    return agg


_agg128 = _make_agg(128)
_agg64 = _make_agg(64)


@functools.partial(
    pl.kernel,
    out_type=jax.ShapeDtypeStruct((NC, NPAD, DDEG), jnp.float32),
    mesh=plsc.VectorSubcoreMesh(**_SC_MESH),
    scratch_types=[
        pltpu.VMEM((CB,), jnp.int32),
        pltpu.VMEM((CB, DDEG), jnp.float32),
        pltpu.VMEM_SHARED((NPAD, DDEG), jnp.float32),
    ],
    compiler_params=_SC_PARAMS,
)
def _deg_sc(ones_hbm, dst_hbm, zero_hbm, out_hbm, dst_v, rows_v, acc_sh):
    c = lax.axis_index("c")
    s = lax.axis_index("s")
    w = c * NS + s
    r0 = s * RPT
    pltpu.sync_copy(zero_hbm, acc_sh.at[pl.ds(r0, RPT)])
    pltpu.sync_copy(ones_hbm, rows_v)
    plsc.subcore_barrier()

    def body(j, carry):
        pltpu.sync_copy(dst_hbm.at[w, j], dst_v)
        pltpu.sync_copy(rows_v, acc_sh.at[dst_v], add=True)
        return carry

    lax.fori_loop(0, CHUNKS, body, 0)
    plsc.subcore_barrier()
    pltpu.sync_copy(acc_sh.at[pl.ds(r0, RPT)],
                    out_hbm.at[c].at[pl.ds(r0, RPT)])


_RB = 1000
_GRID = N // _RB


def _part_spec(core, d):
    return pl.BlockSpec((1, _RB, d), lambda i, c=core: (c, i, 0))


def _row_spec(d):
    return pl.BlockSpec((_RB, d), lambda i: (i, 0))


def _full_spec(shape):
    return pl.BlockSpec(shape, lambda i: tuple(0 for _ in shape))


def _k1_body(d0, d1, x, w1, y_ref, dis_ref):
    deg = d0[0][:, 0:1] + d1[0][:, 0:1] + 1.0
    dis = lax.rsqrt(deg)
    dis_ref[...] = dis
    y_ref[...] = dis * jnp.dot(x[...], w1[...],
                               preferred_element_type=jnp.float32)


def _k2_body(p0, p1, y1, dis, b1, w2, y2_ref):
    h = jnp.maximum(dis[...] * (p0[0] + p1[0] + y1[...]) + b1[...], 0.0)
    y2_ref[...] = dis[...] * jnp.dot(h, w2[...],
                                     preferred_element_type=jnp.float32)


def _k3_body(q0, q1, y2, dis, b2, z_ref):
    h = jnp.maximum(dis[...] * (q0[0] + q1[0] + y2[...]) + b2[...], 0.0)
    z_ref[...] = dis[...] * h


def _k4_body(r0, r1, z, dis, w3, b3, out_ref):
    t = r0[0] + r1[0] + z[...]
    out_ref[...] = dis[...] * jnp.dot(t, w3[...],
                                      preferred_element_type=jnp.float32) + b3[...]


def kernel(x, edge_index, W1, b1, W2, b2, W3, b3):
    f32 = jnp.float32
    src = edge_index[0].reshape(NW, CHUNKS, CB)
    dst = edge_index[1].reshape(NW, CHUNKS, CB)
    onesb = jnp.ones((CB, DDEG), f32)
    zdeg = jnp.zeros((RPT, DDEG), f32)
    z64 = jnp.zeros((RPT, 64), f32)
    z128 = jnp.zeros((RPT, 128), f32)
    b1r = b1.reshape(1, D_H)
    b2r = b2.reshape(1, D_H2)
    b3r = b3.reshape(1, 1)

    degp = _deg_sc(onesb, dst, zdeg)
    y1, dis = pl.pallas_call(
        _k1_body,
        grid=(_GRID,),
        in_specs=[_part_spec(0, DDEG), _part_spec(1, DDEG), _row_spec(D_IN),
                  _full_spec((D_IN, D_H))],
        out_specs=[_row_spec(D_H), _row_spec(1)],
        out_shape=[jax.ShapeDtypeStruct((N, D_H), f32),
                   jax.ShapeDtypeStruct((N, 1), f32)],
    )(degp, degp, x, W1)

    p = _agg128(y1, src, dst, z128)
    y2 = pl.pallas_call(
        _k2_body,
        grid=(_GRID,),
        in_specs=[_part_spec(0, D_H), _part_spec(1, D_H), _row_spec(D_H),
                  _row_spec(1), _full_spec((1, D_H)), _full_spec((D_H, D_H2))],
        out_specs=_row_spec(D_H2),
        out_shape=jax.ShapeDtypeStruct((N, D_H2), f32),
    )(p, p, y1, dis, b1r, W2)

    q = _agg64(y2, src, dst, z64)
    z = pl.pallas_call(
        _k3_body,
        grid=(_GRID,),
        in_specs=[_part_spec(0, D_H2), _part_spec(1, D_H2), _row_spec(D_H2),
                  _row_spec(1), _full_spec((1, D_H2))],
        out_specs=_row_spec(D_H2),
        out_shape=jax.ShapeDtypeStruct((N, D_H2), f32),
    )(q, q, y2, dis, b2r)

    r = _agg64(z, src, dst, z64)
    out = pl.pallas_call(
        _k4_body,
        grid=(_GRID,),
        in_specs=[_part_spec(0, D_H2), _part_spec(1, D_H2), _row_spec(D_H2),
                  _row_spec(1), _full_spec((D_H2, 1)), _full_spec((1, 1))],
        out_specs=_row_spec(1),
        out_shape=jax.ShapeDtypeStruct((N, 1), f32),
    )(r, r, z, dis, W3, b3r)
    return out
